# Initial kernel scaffold; baseline (speedup 1.0000x reference)
import jax
import jax.numpy as jnp
from jax.experimental import pallas as pl


def kernel(x, edge_index, W1_l, b1, W1_r, W2_l, b2, W2_r):
    raise NotImplementedError("write your pallas kernel here")


# speedup vs baseline: 4.5104x; 4.5104x over previous
import functools

import jax
import jax.numpy as jnp
from jax import lax
from jax.experimental import pallas as pl
from jax.experimental.pallas import tpu as pltpu
from jax.experimental.pallas import tpu_sc as plsc

N = 10000
E = 320000
D = 128
DEGW = 16
NC = 2
NS = 16
NW = NC * NS
EPT = E // NW
CH = 80
NCHUNK = EPT // CH
RPT = 624
TAIL = N - NS * RPT


def _zero_rows(ref, nrows, ncol16):
    def body(i, c):
        r = i // ncol16
        k = i % ncol16
        ref[r, pl.ds(k * 16, 16)] = jnp.zeros((16,), jnp.float32)
        return c
    lax.fori_loop(0, nrows * ncol16, body, 0)


def _fill_spmem_zero(zblk, sp, rbase, sid):
    for k in range(7):
        pltpu.sync_copy(zblk, sp.at[pl.ds(rbase + k * CH, CH)])
    pltpu.sync_copy(zblk.at[pl.ds(0, RPT - 7 * CH)],
                    sp.at[pl.ds(rbase + 7 * CH, RPT - 7 * CH)])
    @pl.when(sid == NS - 1)
    def _():
        pltpu.sync_copy(zblk.at[pl.ds(0, TAIL)],
                        sp.at[pl.ds(NS * RPT, TAIL)])


def _sc_feat_body(src_hbm, dst_hbm, y_hbm, acc_out, sidx, didx, rows, acc_sp, sem):
    cid = lax.axis_index("c")
    sid = lax.axis_index("s")
    wid = sid * NC + cid
    rbase = sid * RPT

    _zero_rows(rows, CH, D // 16)
    _fill_spmem_zero(rows, acc_sp, rbase, sid)
    plsc.subcore_barrier()

    @pl.loop(0, NCHUNK)
    def _(j):
        base = wid * EPT + j * CH
        pltpu.sync_copy(src_hbm.at[pl.ds(base, CH)], sidx)
        pltpu.sync_copy(dst_hbm.at[pl.ds(base, CH)], didx)
        pltpu.async_copy(y_hbm.at[sidx], rows, sem).wait()
        pltpu.sync_copy(rows, acc_sp.at[didx], add=True)

    plsc.subcore_barrier()

    pltpu.sync_copy(acc_sp.at[pl.ds(rbase, RPT)],
                    acc_out.at[cid, pl.ds(rbase, RPT)])
    @pl.when(sid == NS - 1)
    def _():
        pltpu.sync_copy(acc_sp.at[pl.ds(NS * RPT, TAIL)],
                        acc_out.at[cid, pl.ds(NS * RPT, TAIL)])


def _make_sc_feat():
    return pl.kernel(
        _sc_feat_body,
        out_type=jax.ShapeDtypeStruct((NC, N, D), jnp.float32),
        scratch_types=[
            pltpu.VMEM((CH,), jnp.int32),
            pltpu.VMEM((CH,), jnp.int32),
            pltpu.VMEM((CH, D), jnp.float32),
            pltpu.VMEM_SHARED((N, D), jnp.float32),
            pltpu.SemaphoreType.DMA,
        ],
        mesh=plsc.VectorSubcoreMesh(core_axis_name="c", subcore_axis_name="s"),
    )


def _sc_deg_body(dst_hbm, deg_out, didx, ones, deg_sp):
    cid = lax.axis_index("c")
    sid = lax.axis_index("s")
    wid = sid * NC + cid
    rbase = sid * RPT

    _zero_rows(ones, CH, D // 16)
    _fill_spmem_zero(ones, deg_sp, rbase, sid)
    def fill1(i, c):
        r = i // (D // 16)
        k = i % (D // 16)
        ones[r, pl.ds(k * 16, 16)] = jnp.ones((16,), jnp.float32)
        return c
    lax.fori_loop(0, CH * (D // 16), fill1, 0)
    plsc.subcore_barrier()

    @pl.loop(0, NCHUNK)
    def _(j):
        base = wid * EPT + j * CH
        pltpu.sync_copy(dst_hbm.at[pl.ds(base, CH)], didx)
        pltpu.sync_copy(ones, deg_sp.at[didx], add=True)

    plsc.subcore_barrier()
    pltpu.sync_copy(deg_sp.at[pl.ds(rbase, RPT)],
                    deg_out.at[cid, pl.ds(rbase, RPT)])
    @pl.when(sid == NS - 1)
    def _():
        pltpu.sync_copy(deg_sp.at[pl.ds(NS * RPT, TAIL)],
                        deg_out.at[cid, pl.ds(NS * RPT, TAIL)])


def _make_sc_deg():
    return pl.kernel(
        _sc_deg_body,
        out_type=jax.ShapeDtypeStruct((NC, N, D), jnp.float32),
        scratch_types=[
            pltpu.VMEM((CH,), jnp.int32),
            pltpu.VMEM((CH, D), jnp.float32),
            pltpu.VMEM_SHARED((N, D), jnp.float32),
        ],
        mesh=plsc.VectorSubcoreMesh(core_axis_name="c", subcore_axis_name="s"),
    )


BLK = 400


def _mm2_body(x_ref, wlT_ref, wrT_ref, b_ref, y_ref, z_ref):
    xb = x_ref[...]
    y_ref[...] = jnp.dot(xb, wlT_ref[...], preferred_element_type=jnp.float32)
    z_ref[...] = (jnp.dot(xb, wrT_ref[...], preferred_element_type=jnp.float32)
                  + b_ref[...])


def _mm2(x, WlT, WrT, b):
    return pl.pallas_call(
        _mm2_body,
        grid=(N // BLK,),
        in_specs=[
            pl.BlockSpec((BLK, D), lambda i: (i, 0)),
            pl.BlockSpec((D, D), lambda i: (0, 0)),
            pl.BlockSpec((D, D), lambda i: (0, 0)),
            pl.BlockSpec((1, D), lambda i: (0, 0)),
        ],
        out_specs=[pl.BlockSpec((BLK, D), lambda i: (i, 0)),
                   pl.BlockSpec((BLK, D), lambda i: (i, 0))],
        out_shape=[jax.ShapeDtypeStruct((N, D), jnp.float32),
                   jax.ShapeDtypeStruct((N, D), jnp.float32)],
    )(x, WlT, WrT, b.reshape(1, D))


def _combine_body(relu, acc_ref, deg_ref, z_ref, o_ref):
    a = acc_ref[0] + acc_ref[1]
    dg = deg_ref[0] + deg_ref[1]
    d = dg[:, 0:1]
    h = a / jnp.maximum(d, 1.0) + z_ref[...]
    if relu:
        h = jnp.maximum(h, 0.0)
    o_ref[...] = h


def _combine(acc, deg, z, relu):
    return pl.pallas_call(
        functools.partial(_combine_body, relu),
        grid=(N // BLK,),
        in_specs=[
            pl.BlockSpec((NC, BLK, D), lambda i: (0, i, 0)),
            pl.BlockSpec((NC, BLK, D), lambda i: (0, i, 0)),
            pl.BlockSpec((BLK, D), lambda i: (i, 0)),
        ],
        out_specs=pl.BlockSpec((BLK, D), lambda i: (i, 0)),
        out_shape=jax.ShapeDtypeStruct((N, D), jnp.float32),
    )(acc, deg, z)


def kernel(x, edge_index, W1_l, b1, W1_r, W2_l, b2, W2_r):
    src = edge_index[0]
    dst = edge_index[1]

    sc_feat = _make_sc_feat()
    sc_deg = _make_sc_deg()

    deg = sc_deg(dst)
    y1, z1 = _mm2(x, W1_l.T, W1_r.T, b1)
    acc1 = sc_feat(src, dst, y1)
    h = _combine(acc1, deg, z1, True)

    y2, z2 = _mm2(h, W2_l.T, W2_r.T, b2)
    acc2 = sc_feat(src, dst, y2)
    out = _combine(acc2, deg, z2, False)
    return out


# speedup vs baseline: 7.7864x; 1.7263x over previous
import functools

import jax
import jax.numpy as jnp
from jax import lax
from jax.experimental import pallas as pl
from jax.experimental.pallas import tpu as pltpu
from jax.experimental.pallas import tpu_sc as plsc

N = 10000
E = 320000
D = 128
DEGW = 16
NC = 2
NS = 16
NW = NC * NS
EPT = E // NW
CH = 80
NCHUNK = EPT // CH
RPT = 624
TAIL = N - NS * RPT


def _zero_rows(ref, nrows, ncol16):
    def body(i, c):
        r = i // ncol16
        k = i % ncol16
        ref[r, pl.ds(k * 16, 16)] = jnp.zeros((16,), jnp.float32)
        return c
    lax.fori_loop(0, nrows * ncol16, body, 0)


def _fill_spmem_zero(zblk, sp, rbase, sid):
    nfull = RPT // CH
    rem = RPT - nfull * CH
    for k in range(nfull):
        pltpu.sync_copy(zblk, sp.at[pl.ds(rbase + k * CH, CH)])
    if rem:
        pltpu.sync_copy(zblk.at[pl.ds(0, rem)],
                        sp.at[pl.ds(rbase + nfull * CH, rem)])
    @pl.when(sid == NS - 1)
    def _():
        pltpu.sync_copy(zblk.at[pl.ds(0, TAIL)],
                        sp.at[pl.ds(NS * RPT, TAIL)])


def _sc_feat_body(src_hbm, dst_hbm, y_hbm, acc_out,
                  sidx_a, sidx_b, didx_a, didx_b,
                  rows_a, rows_b, acc_sp, semg, semsi, semdi):
    cid = lax.axis_index("c")
    sid = lax.axis_index("s")
    wid = sid * NC + cid
    rbase = sid * RPT
    ebase = wid * EPT

    _zero_rows(rows_a, CH, D // 16)
    _fill_spmem_zero(rows_a, acc_sp, rbase, sid)
    plsc.subcore_barrier()

    rows = (rows_a, rows_b)
    sib = (sidx_a, sidx_b)
    dib = (didx_a, didx_b)

    pltpu.sync_copy(src_hbm.at[pl.ds(ebase, CH)], sidx_a)
    pltpu.async_copy(y_hbm.at[sidx_a], rows_a, semg)
    pltpu.async_copy(src_hbm.at[pl.ds(ebase + CH, CH)], sidx_b, semsi)
    pltpu.async_copy(dst_hbm.at[pl.ds(ebase, CH)], didx_a, semdi)

    @pl.loop(0, NCHUNK, step=2)
    def _(j):
        for b in range(2):
            jj = j + b

            @pl.when(jj < NCHUNK)
            def _():
                pltpu.make_async_copy(y_hbm.at[sib[b]], rows[b], semg).wait()

                @pl.when(jj + 1 < NCHUNK)
                def _():
                    pltpu.make_async_copy(
                        src_hbm.at[pl.ds(ebase + (jj + 1) * CH, CH)],
                        sib[1 - b], semsi).wait()
                    pltpu.async_copy(y_hbm.at[sib[1 - b]], rows[1 - b], semg)

                    @pl.when(jj + 2 < NCHUNK)
                    def _():
                        pltpu.async_copy(
                            src_hbm.at[pl.ds(ebase + (jj + 2) * CH, CH)],
                            sib[b], semsi)

                    pltpu.async_copy(
                        dst_hbm.at[pl.ds(ebase + (jj + 1) * CH, CH)],
                        dib[1 - b], semdi)

                pltpu.make_async_copy(
                    dst_hbm.at[pl.ds(ebase + jj * CH, CH)],
                    dib[b], semdi).wait()
                pltpu.sync_copy(rows[b], acc_sp.at[dib[b]], add=True)

    plsc.subcore_barrier()

    pltpu.sync_copy(acc_sp.at[pl.ds(rbase, RPT)],
                    acc_out.at[cid, pl.ds(rbase, RPT)])
    @pl.when(sid == NS - 1)
    def _():
        pltpu.sync_copy(acc_sp.at[pl.ds(NS * RPT, TAIL)],
                        acc_out.at[cid, pl.ds(NS * RPT, TAIL)])


def _make_sc_feat():
    return pl.kernel(
        _sc_feat_body,
        out_type=jax.ShapeDtypeStruct((NC, N, D), jnp.float32),
        scratch_types=[
            pltpu.VMEM((CH,), jnp.int32),
            pltpu.VMEM((CH,), jnp.int32),
            pltpu.VMEM((CH,), jnp.int32),
            pltpu.VMEM((CH,), jnp.int32),
            pltpu.VMEM((CH, D), jnp.float32),
            pltpu.VMEM((CH, D), jnp.float32),
            pltpu.VMEM_SHARED((N, D), jnp.float32),
            pltpu.SemaphoreType.DMA,
            pltpu.SemaphoreType.DMA,
            pltpu.SemaphoreType.DMA,
        ],
        mesh=plsc.VectorSubcoreMesh(core_axis_name="c", subcore_axis_name="s"),
    )


def _sc_deg_body(dst_hbm, deg_out, didx_all, ones, deg_sp):
    cid = lax.axis_index("c")
    sid = lax.axis_index("s")
    wid = sid * NC + cid
    rbase = sid * RPT

    _zero_rows(ones, CH, D // 16)
    _fill_spmem_zero(ones, deg_sp, rbase, sid)
    def fill1(i, c):
        r = i // (D // 16)
        k = i % (D // 16)
        ones[r, pl.ds(k * 16, 16)] = jnp.ones((16,), jnp.float32)
        return c
    lax.fori_loop(0, CH * (D // 16), fill1, 0)
    pltpu.sync_copy(dst_hbm.at[wid], didx_all)
    plsc.subcore_barrier()

    @pl.loop(0, NCHUNK)
    def _(j):
        pltpu.sync_copy(ones, deg_sp.at[didx_all.at[j]], add=True)

    plsc.subcore_barrier()
    pltpu.sync_copy(deg_sp.at[pl.ds(rbase, RPT)],
                    deg_out.at[cid, pl.ds(rbase, RPT)])
    @pl.when(sid == NS - 1)
    def _():
        pltpu.sync_copy(deg_sp.at[pl.ds(NS * RPT, TAIL)],
                        deg_out.at[cid, pl.ds(NS * RPT, TAIL)])


def _make_sc_deg():
    return pl.kernel(
        _sc_deg_body,
        out_type=jax.ShapeDtypeStruct((NC, N, D), jnp.float32),
        scratch_types=[
            pltpu.VMEM((NCHUNK, CH), jnp.int32),
            pltpu.VMEM((CH, D), jnp.float32),
            pltpu.VMEM_SHARED((N, D), jnp.float32),
        ],
        mesh=plsc.VectorSubcoreMesh(core_axis_name="c", subcore_axis_name="s"),
    )


BLK = 400


def _mm2_body(x_ref, wlT_ref, wrT_ref, b_ref, y_ref, z_ref):
    xb = x_ref[...]
    y_ref[...] = jnp.dot(xb, wlT_ref[...], preferred_element_type=jnp.float32)
    z_ref[...] = (jnp.dot(xb, wrT_ref[...], preferred_element_type=jnp.float32)
                  + b_ref[...])


def _mm2(x, WlT, WrT, b):
    return pl.pallas_call(
        _mm2_body,
        grid=(N // BLK,),
        in_specs=[
            pl.BlockSpec((BLK, D), lambda i: (i, 0)),
            pl.BlockSpec((D, D), lambda i: (0, 0)),
            pl.BlockSpec((D, D), lambda i: (0, 0)),
            pl.BlockSpec((1, D), lambda i: (0, 0)),
        ],
        out_specs=[pl.BlockSpec((BLK, D), lambda i: (i, 0)),
                   pl.BlockSpec((BLK, D), lambda i: (i, 0))],
        out_shape=[jax.ShapeDtypeStruct((N, D), jnp.float32),
                   jax.ShapeDtypeStruct((N, D), jnp.float32)],
    )(x, WlT, WrT, b.reshape(1, D))


def _combine_body(relu, acc_ref, deg_ref, z_ref, o_ref):
    a = acc_ref[0] + acc_ref[1]
    dg = deg_ref[0] + deg_ref[1]
    d = dg[:, 0:1]
    h = a / jnp.maximum(d, 1.0) + z_ref[...]
    if relu:
        h = jnp.maximum(h, 0.0)
    o_ref[...] = h


def _combine(acc, deg, z, relu):
    return pl.pallas_call(
        functools.partial(_combine_body, relu),
        grid=(N // BLK,),
        in_specs=[
            pl.BlockSpec((NC, BLK, D), lambda i: (0, i, 0)),
            pl.BlockSpec((NC, BLK, D), lambda i: (0, i, 0)),
            pl.BlockSpec((BLK, D), lambda i: (i, 0)),
        ],
        out_specs=pl.BlockSpec((BLK, D), lambda i: (i, 0)),
        out_shape=jax.ShapeDtypeStruct((N, D), jnp.float32),
    )(acc, deg, z)


def kernel(x, edge_index, W1_l, b1, W1_r, W2_l, b2, W2_r):
    src = edge_index[0]
    dst = edge_index[1]
    dst3 = dst.reshape(NW, NCHUNK, CH)

    sc_feat = _make_sc_feat()
    sc_deg = _make_sc_deg()

    deg = sc_deg(dst3)
    y1, z1 = _mm2(x, W1_l.T, W1_r.T, b1)
    acc1 = sc_feat(src, dst, y1)
    h = _combine(acc1, deg, z1, True)

    y2, z2 = _mm2(h, W2_l.T, W2_r.T, b2)
    acc2 = sc_feat(src, dst, y2)
    out = _combine(acc2, deg, z2, False)
    return out


# speedup vs baseline: 8.0502x; 1.0339x over previous
import functools

import jax
import jax.numpy as jnp
from jax import lax
from jax.experimental import pallas as pl
from jax.experimental.pallas import tpu as pltpu
from jax.experimental.pallas import tpu_sc as plsc

N = 10000
E = 320000
D = 128
DEGW = 16
NC = 2
NS = 16
NW = NC * NS
EPT = E // NW
CH = 80
NCHUNK = EPT // CH
RPT = 624
TAIL = N - NS * RPT


def _zero_rows(ref, nrows, ncol16):
    def body(i, c):
        r = i // ncol16
        k = i % ncol16
        ref[r, pl.ds(k * 16, 16)] = jnp.zeros((16,), jnp.float32)
        return c
    lax.fori_loop(0, nrows * ncol16, body, 0)


def _fill_spmem_zero(zblk, sp, rbase, sid):
    nfull = RPT // CH
    rem = RPT - nfull * CH
    for k in range(nfull):
        pltpu.sync_copy(zblk, sp.at[pl.ds(rbase + k * CH, CH)])
    if rem:
        pltpu.sync_copy(zblk.at[pl.ds(0, rem)],
                        sp.at[pl.ds(rbase + nfull * CH, rem)])
    @pl.when(sid == NS - 1)
    def _():
        pltpu.sync_copy(zblk.at[pl.ds(0, TAIL)],
                        sp.at[pl.ds(NS * RPT, TAIL)])


def _sc_feat_body(src_hbm, dst_hbm, y_hbm, acc_out,
                  sidx_a, sidx_b, didx_a, didx_b,
                  rows_a, rows_b, acc_sp, semg, semsi, semdi, semsc):
    cid = lax.axis_index("c")
    sid = lax.axis_index("s")
    wid = sid * NC + cid
    rbase = sid * RPT
    ebase = wid * EPT

    _zero_rows(rows_a, CH, D // 16)
    _fill_spmem_zero(rows_a, acc_sp, rbase, sid)
    plsc.subcore_barrier()

    rows = (rows_a, rows_b)
    sib = (sidx_a, sidx_b)
    dib = (didx_a, didx_b)

    pltpu.sync_copy(src_hbm.at[pl.ds(ebase, CH)], sidx_a)
    pltpu.async_copy(y_hbm.at[sidx_a], rows_a, semg)
    pltpu.async_copy(src_hbm.at[pl.ds(ebase + CH, CH)], sidx_b, semsi)
    pltpu.async_copy(dst_hbm.at[pl.ds(ebase, CH)], didx_a, semdi)

    @pl.loop(0, NCHUNK, step=2)
    def _(j):
        for b in range(2):
            jj = j + b

            @pl.when(jj < NCHUNK)
            def _():
                pltpu.make_async_copy(y_hbm.at[sib[b]], rows[b], semg).wait()

                if b == 0:
                    @pl.when(jj > 0)
                    def _():
                        pltpu.make_async_copy(
                            rows[1 - b], acc_sp.at[dib[1 - b]], semsc).wait()
                else:
                    pltpu.make_async_copy(
                        rows[1 - b], acc_sp.at[dib[1 - b]], semsc).wait()

                @pl.when(jj + 1 < NCHUNK)
                def _():
                    pltpu.make_async_copy(
                        src_hbm.at[pl.ds(ebase + (jj + 1) * CH, CH)],
                        sib[1 - b], semsi).wait()
                    pltpu.async_copy(y_hbm.at[sib[1 - b]], rows[1 - b], semg)

                    @pl.when(jj + 2 < NCHUNK)
                    def _():
                        pltpu.async_copy(
                            src_hbm.at[pl.ds(ebase + (jj + 2) * CH, CH)],
                            sib[b], semsi)

                    pltpu.async_copy(
                        dst_hbm.at[pl.ds(ebase + (jj + 1) * CH, CH)],
                        dib[1 - b], semdi)

                pltpu.make_async_copy(
                    dst_hbm.at[pl.ds(ebase + jj * CH, CH)],
                    dib[b], semdi).wait()
                pltpu.async_copy(rows[b], acc_sp.at[dib[b]], semsc, add=True)

    pltpu.make_async_copy(
        rows[(NCHUNK - 1) % 2], acc_sp.at[dib[(NCHUNK - 1) % 2]], semsc).wait()
    plsc.subcore_barrier()

    pltpu.sync_copy(acc_sp.at[pl.ds(rbase, RPT)],
                    acc_out.at[cid, pl.ds(rbase, RPT)])
    @pl.when(sid == NS - 1)
    def _():
        pltpu.sync_copy(acc_sp.at[pl.ds(NS * RPT, TAIL)],
                        acc_out.at[cid, pl.ds(NS * RPT, TAIL)])


def _make_sc_feat():
    return pl.kernel(
        _sc_feat_body,
        out_type=jax.ShapeDtypeStruct((NC, N, D), jnp.float32),
        scratch_types=[
            pltpu.VMEM((CH,), jnp.int32),
            pltpu.VMEM((CH,), jnp.int32),
            pltpu.VMEM((CH,), jnp.int32),
            pltpu.VMEM((CH,), jnp.int32),
            pltpu.VMEM((CH, D), jnp.float32),
            pltpu.VMEM((CH, D), jnp.float32),
            pltpu.VMEM_SHARED((N, D), jnp.float32),
            pltpu.SemaphoreType.DMA,
            pltpu.SemaphoreType.DMA,
            pltpu.SemaphoreType.DMA,
            pltpu.SemaphoreType.DMA,
        ],
        mesh=plsc.VectorSubcoreMesh(core_axis_name="c", subcore_axis_name="s"),
    )


def _sc_deg_body(dst_hbm, deg_out, didx_all, ones, deg_sp, semsc):
    cid = lax.axis_index("c")
    sid = lax.axis_index("s")
    wid = sid * NC + cid
    rbase = sid * RPT

    _zero_rows(ones, CH, D // 16)
    _fill_spmem_zero(ones, deg_sp, rbase, sid)
    def fill1(i, c):
        r = i // (D // 16)
        k = i % (D // 16)
        ones[r, pl.ds(k * 16, 16)] = jnp.ones((16,), jnp.float32)
        return c
    lax.fori_loop(0, CH * (D // 16), fill1, 0)
    pltpu.sync_copy(dst_hbm.at[wid], didx_all)
    plsc.subcore_barrier()

    @pl.loop(0, NCHUNK)
    def _(j):
        @pl.when(j >= 4)
        def _():
            pltpu.make_async_copy(ones, deg_sp.at[didx_all.at[j - 4]],
                                  semsc).wait()
        pltpu.async_copy(ones, deg_sp.at[didx_all.at[j]], semsc, add=True)

    for k in range(4):
        pltpu.make_async_copy(ones, deg_sp.at[didx_all.at[NCHUNK - 4 + k]],
                              semsc).wait()
    plsc.subcore_barrier()
    pltpu.sync_copy(deg_sp.at[pl.ds(rbase, RPT)],
                    deg_out.at[cid, pl.ds(rbase, RPT)])
    @pl.when(sid == NS - 1)
    def _():
        pltpu.sync_copy(deg_sp.at[pl.ds(NS * RPT, TAIL)],
                        deg_out.at[cid, pl.ds(NS * RPT, TAIL)])


def _make_sc_deg():
    return pl.kernel(
        _sc_deg_body,
        out_type=jax.ShapeDtypeStruct((NC, N, D), jnp.float32),
        scratch_types=[
            pltpu.VMEM((NCHUNK, CH), jnp.int32),
            pltpu.VMEM((CH, D), jnp.float32),
            pltpu.VMEM_SHARED((N, D), jnp.float32),
            pltpu.SemaphoreType.DMA,
        ],
        mesh=plsc.VectorSubcoreMesh(core_axis_name="c", subcore_axis_name="s"),
    )


BLK = 400


def _mm2_body(x_ref, wlT_ref, wrT_ref, b_ref, y_ref, z_ref):
    xb = x_ref[...]
    y_ref[...] = jnp.dot(xb, wlT_ref[...], preferred_element_type=jnp.float32)
    z_ref[...] = (jnp.dot(xb, wrT_ref[...], preferred_element_type=jnp.float32)
                  + b_ref[...])


def _mm2(x, WlT, WrT, b):
    return pl.pallas_call(
        _mm2_body,
        grid=(N // BLK,),
        in_specs=[
            pl.BlockSpec((BLK, D), lambda i: (i, 0)),
            pl.BlockSpec((D, D), lambda i: (0, 0)),
            pl.BlockSpec((D, D), lambda i: (0, 0)),
            pl.BlockSpec((1, D), lambda i: (0, 0)),
        ],
        out_specs=[pl.BlockSpec((BLK, D), lambda i: (i, 0)),
                   pl.BlockSpec((BLK, D), lambda i: (i, 0))],
        out_shape=[jax.ShapeDtypeStruct((N, D), jnp.float32),
                   jax.ShapeDtypeStruct((N, D), jnp.float32)],
    )(x, WlT, WrT, b.reshape(1, D))


def _combine_mm_body(acc_ref, deg_ref, z_ref, wlT_ref, wrT_ref, b_ref,
                     y_ref, z2_ref):
    a = acc_ref[0] + acc_ref[1]
    dg = deg_ref[0] + deg_ref[1]
    h = a / jnp.maximum(dg[:, 0:1], 1.0) + z_ref[...]
    h = jnp.maximum(h, 0.0)
    y_ref[...] = jnp.dot(h, wlT_ref[...], preferred_element_type=jnp.float32)
    z2_ref[...] = (jnp.dot(h, wrT_ref[...], preferred_element_type=jnp.float32)
                   + b_ref[...])


def _combine_mm(acc, deg, z, WlT, WrT, b):
    return pl.pallas_call(
        _combine_mm_body,
        grid=(N // BLK,),
        in_specs=[
            pl.BlockSpec((NC, BLK, D), lambda i: (0, i, 0)),
            pl.BlockSpec((NC, BLK, D), lambda i: (0, i, 0)),
            pl.BlockSpec((BLK, D), lambda i: (i, 0)),
            pl.BlockSpec((D, D), lambda i: (0, 0)),
            pl.BlockSpec((D, D), lambda i: (0, 0)),
            pl.BlockSpec((1, D), lambda i: (0, 0)),
        ],
        out_specs=[pl.BlockSpec((BLK, D), lambda i: (i, 0)),
                   pl.BlockSpec((BLK, D), lambda i: (i, 0))],
        out_shape=[jax.ShapeDtypeStruct((N, D), jnp.float32),
                   jax.ShapeDtypeStruct((N, D), jnp.float32)],
    )(acc, deg, z, WlT, WrT, b.reshape(1, D))


def _combine_body(relu, acc_ref, deg_ref, z_ref, o_ref):
    a = acc_ref[0] + acc_ref[1]
    dg = deg_ref[0] + deg_ref[1]
    d = dg[:, 0:1]
    h = a / jnp.maximum(d, 1.0) + z_ref[...]
    if relu:
        h = jnp.maximum(h, 0.0)
    o_ref[...] = h


def _combine(acc, deg, z, relu):
    return pl.pallas_call(
        functools.partial(_combine_body, relu),
        grid=(N // BLK,),
        in_specs=[
            pl.BlockSpec((NC, BLK, D), lambda i: (0, i, 0)),
            pl.BlockSpec((NC, BLK, D), lambda i: (0, i, 0)),
            pl.BlockSpec((BLK, D), lambda i: (i, 0)),
        ],
        out_specs=pl.BlockSpec((BLK, D), lambda i: (i, 0)),
        out_shape=jax.ShapeDtypeStruct((N, D), jnp.float32),
    )(acc, deg, z)


def kernel(x, edge_index, W1_l, b1, W1_r, W2_l, b2, W2_r):
    src = edge_index[0]
    dst = edge_index[1]
    dst3 = dst.reshape(NW, NCHUNK, CH)

    sc_feat = _make_sc_feat()
    sc_deg = _make_sc_deg()

    deg = sc_deg(dst3)
    y1, z1 = _mm2(x, W1_l.T, W1_r.T, b1)
    acc1 = sc_feat(src, dst, y1)
    y2, z2 = _combine_mm(acc1, deg, z1, W2_l.T, W2_r.T, b2)
    acc2 = sc_feat(src, dst, y2)
    out = _combine(acc2, deg, z2, False)
    return out


# speedup vs baseline: 10.4382x; 1.2966x over previous
import functools

import jax
import jax.numpy as jnp
from jax import lax
from jax.experimental import pallas as pl
from jax.experimental.pallas import tpu as pltpu
from jax.experimental.pallas import tpu_sc as plsc

N = 10000
E = 320000
D = 128
DEGW = 16
NC = 2
NS = 16
NW = NC * NS
EPT = E // NW
CH = 80
NCHUNK = EPT // CH
RPT = 624
TAIL = N - NS * RPT


def _zero_rows(ref, nrows, ncol16):
    def body(i, c):
        r = i // ncol16
        k = i % ncol16
        ref[r, pl.ds(k * 16, 16)] = jnp.zeros((16,), jnp.float32)
        return c
    lax.fori_loop(0, nrows * ncol16, body, 0)


def _fill_spmem_zero(zblk, sp, rbase, sid):
    nfull = RPT // CH
    rem = RPT - nfull * CH
    for k in range(nfull):
        pltpu.sync_copy(zblk, sp.at[pl.ds(rbase + k * CH, CH)])
    if rem:
        pltpu.sync_copy(zblk.at[pl.ds(0, rem)],
                        sp.at[pl.ds(rbase + nfull * CH, rem)])
    @pl.when(sid == NS - 1)
    def _():
        pltpu.sync_copy(zblk.at[pl.ds(0, TAIL)],
                        sp.at[pl.ds(NS * RPT, TAIL)])


def _sc_feat_body(src_hbm, dst_hbm, y_hbm, acc_out,
                  sidx_a, sidx_b, sidx_c, didx_a, didx_b, didx_c,
                  rows_a, rows_b, rows_c, acc_sp, semg, semsi, semdi, semsc):
    cid = lax.axis_index("c")
    sid = lax.axis_index("s")
    wid = sid * NC + cid
    rbase = sid * RPT
    ebase = wid * EPT

    _zero_rows(rows_a, CH, D // 16)
    _fill_spmem_zero(rows_a, acc_sp, rbase, sid)
    plsc.subcore_barrier()

    rows = (rows_a, rows_b, rows_c)
    sib = (sidx_a, sidx_b, sidx_c)
    dib = (didx_a, didx_b, didx_c)

    def si_load(jj, sl):
        return pltpu.async_copy(
            src_hbm.at[pl.ds(ebase + jj * CH, CH)], sl, semsi)

    def di_load(jj, dl):
        return pltpu.async_copy(
            dst_hbm.at[pl.ds(ebase + jj * CH, CH)], dl, semdi)

    pltpu.sync_copy(src_hbm.at[pl.ds(ebase, CH)], sidx_a)
    pltpu.async_copy(y_hbm.at[sidx_a], rows_a, semg)
    pltpu.sync_copy(src_hbm.at[pl.ds(ebase + CH, CH)], sidx_b)
    pltpu.async_copy(y_hbm.at[sidx_b], rows_b, semg)
    si_load(2, sidx_c)
    di_load(0, didx_a)

    @pl.loop(0, NCHUNK + 2 - ((NCHUNK + 2) % 3), step=3)
    def _(j):
        for b0 in range(3):
            jj = j + b0
            b = b0

            @pl.when(jj < NCHUNK)
            def _():
                pltpu.make_async_copy(y_hbm.at[sib[b]], rows[b], semg).wait()

                def wait_sc_prev():
                    pltpu.make_async_copy(
                        rows[(b + 2) % 3], acc_sp.at[dib[(b + 2) % 3]],
                        semsc).wait()
                if b0 == 0:
                    @pl.when(jj > 0)
                    def _():
                        wait_sc_prev()
                else:
                    wait_sc_prev()

                @pl.when(jj + 2 < NCHUNK)
                def _():
                    pltpu.make_async_copy(
                        src_hbm.at[pl.ds(ebase + (jj + 2) * CH, CH)],
                        sib[(b + 2) % 3], semsi).wait()
                    pltpu.async_copy(y_hbm.at[sib[(b + 2) % 3]],
                                     rows[(b + 2) % 3], semg)

                    @pl.when(jj + 3 < NCHUNK)
                    def _():
                        si_load(jj + 3, sib[b])

                @pl.when(jj + 1 < NCHUNK)
                def _():
                    di_load(jj + 1, dib[(b + 1) % 3])

                pltpu.make_async_copy(
                    dst_hbm.at[pl.ds(ebase + jj * CH, CH)],
                    dib[b], semdi).wait()
                pltpu.async_copy(rows[b], acc_sp.at[dib[b]], semsc, add=True)

    pltpu.make_async_copy(
        rows[(NCHUNK - 1) % 3], acc_sp.at[dib[(NCHUNK - 1) % 3]], semsc).wait()
    plsc.subcore_barrier()

    pltpu.sync_copy(acc_sp.at[pl.ds(rbase, RPT)],
                    acc_out.at[cid, pl.ds(rbase, RPT)])
    @pl.when(sid == NS - 1)
    def _():
        pltpu.sync_copy(acc_sp.at[pl.ds(NS * RPT, TAIL)],
                        acc_out.at[cid, pl.ds(NS * RPT, TAIL)])


def _make_sc_feat():
    return pl.kernel(
        _sc_feat_body,
        out_type=jax.ShapeDtypeStruct((NC, N, D), jnp.float32),
        scratch_types=[
            pltpu.VMEM((CH,), jnp.int32),
            pltpu.VMEM((CH,), jnp.int32),
            pltpu.VMEM((CH,), jnp.int32),
            pltpu.VMEM((CH,), jnp.int32),
            pltpu.VMEM((CH,), jnp.int32),
            pltpu.VMEM((CH,), jnp.int32),
            pltpu.VMEM((CH, D), jnp.float32),
            pltpu.VMEM((CH, D), jnp.float32),
            pltpu.VMEM((CH, D), jnp.float32),
            pltpu.VMEM_SHARED((N, D), jnp.float32),
            pltpu.SemaphoreType.DMA,
            pltpu.SemaphoreType.DMA,
            pltpu.SemaphoreType.DMA,
            pltpu.SemaphoreType.DMA,
        ],
        mesh=plsc.VectorSubcoreMesh(core_axis_name="c", subcore_axis_name="s"),
    )


def _sc_deg_body(dst_hbm, deg_out, didx_all, ones, deg_sp, semsc):
    cid = lax.axis_index("c")
    sid = lax.axis_index("s")
    wid = sid * NC + cid
    rbase = sid * RPT

    _zero_rows(ones, CH, D // 16)
    _fill_spmem_zero(ones, deg_sp, rbase, sid)
    def fill1(i, c):
        r = i // (D // 16)
        k = i % (D // 16)
        ones[r, pl.ds(k * 16, 16)] = jnp.ones((16,), jnp.float32)
        return c
    lax.fori_loop(0, CH * (D // 16), fill1, 0)
    pltpu.sync_copy(dst_hbm.at[wid], didx_all)
    plsc.subcore_barrier()

    @pl.loop(0, NCHUNK)
    def _(j):
        @pl.when(j >= 4)
        def _():
            pltpu.make_async_copy(ones, deg_sp.at[didx_all.at[j - 4]],
                                  semsc).wait()
        pltpu.async_copy(ones, deg_sp.at[didx_all.at[j]], semsc, add=True)

    for k in range(4):
        pltpu.make_async_copy(ones, deg_sp.at[didx_all.at[NCHUNK - 4 + k]],
                              semsc).wait()
    plsc.subcore_barrier()
    pltpu.sync_copy(deg_sp.at[pl.ds(rbase, RPT)],
                    deg_out.at[cid, pl.ds(rbase, RPT)])
    @pl.when(sid == NS - 1)
    def _():
        pltpu.sync_copy(deg_sp.at[pl.ds(NS * RPT, TAIL)],
                        deg_out.at[cid, pl.ds(NS * RPT, TAIL)])


def _make_sc_deg():
    return pl.kernel(
        _sc_deg_body,
        out_type=jax.ShapeDtypeStruct((NC, N, D), jnp.float32),
        scratch_types=[
            pltpu.VMEM((NCHUNK, CH), jnp.int32),
            pltpu.VMEM((CH, D), jnp.float32),
            pltpu.VMEM_SHARED((N, D), jnp.float32),
            pltpu.SemaphoreType.DMA,
        ],
        mesh=plsc.VectorSubcoreMesh(core_axis_name="c", subcore_axis_name="s"),
    )


BLK = 400


def _mm2_body(x_ref, wlT_ref, wrT_ref, b_ref, y_ref, z_ref):
    xb = x_ref[...]
    y_ref[...] = jnp.dot(xb, wlT_ref[...], preferred_element_type=jnp.float32)
    z_ref[...] = (jnp.dot(xb, wrT_ref[...], preferred_element_type=jnp.float32)
                  + b_ref[...])


def _mm2(x, WlT, WrT, b):
    return pl.pallas_call(
        _mm2_body,
        grid=(N // BLK,),
        in_specs=[
            pl.BlockSpec((BLK, D), lambda i: (i, 0)),
            pl.BlockSpec((D, D), lambda i: (0, 0)),
            pl.BlockSpec((D, D), lambda i: (0, 0)),
            pl.BlockSpec((1, D), lambda i: (0, 0)),
        ],
        out_specs=[pl.BlockSpec((BLK, D), lambda i: (i, 0)),
                   pl.BlockSpec((BLK, D), lambda i: (i, 0))],
        out_shape=[jax.ShapeDtypeStruct((N, D), jnp.float32),
                   jax.ShapeDtypeStruct((N, D), jnp.float32)],
    )(x, WlT, WrT, b.reshape(1, D))


def _combine_mm_body(acc_ref, deg_ref, z_ref, wlT_ref, wrT_ref, b_ref,
                     y_ref, z2_ref):
    a = acc_ref[0] + acc_ref[1]
    dg = deg_ref[0] + deg_ref[1]
    h = a / jnp.maximum(dg[:, 0:1], 1.0) + z_ref[...]
    h = jnp.maximum(h, 0.0)
    y_ref[...] = jnp.dot(h, wlT_ref[...], preferred_element_type=jnp.float32)
    z2_ref[...] = (jnp.dot(h, wrT_ref[...], preferred_element_type=jnp.float32)
                   + b_ref[...])


def _combine_mm(acc, deg, z, WlT, WrT, b):
    return pl.pallas_call(
        _combine_mm_body,
        grid=(N // BLK,),
        in_specs=[
            pl.BlockSpec((NC, BLK, D), lambda i: (0, i, 0)),
            pl.BlockSpec((NC, BLK, D), lambda i: (0, i, 0)),
            pl.BlockSpec((BLK, D), lambda i: (i, 0)),
            pl.BlockSpec((D, D), lambda i: (0, 0)),
            pl.BlockSpec((D, D), lambda i: (0, 0)),
            pl.BlockSpec((1, D), lambda i: (0, 0)),
        ],
        out_specs=[pl.BlockSpec((BLK, D), lambda i: (i, 0)),
                   pl.BlockSpec((BLK, D), lambda i: (i, 0))],
        out_shape=[jax.ShapeDtypeStruct((N, D), jnp.float32),
                   jax.ShapeDtypeStruct((N, D), jnp.float32)],
    )(acc, deg, z, WlT, WrT, b.reshape(1, D))


def _combine_body(relu, acc_ref, deg_ref, z_ref, o_ref):
    a = acc_ref[0] + acc_ref[1]
    dg = deg_ref[0] + deg_ref[1]
    d = dg[:, 0:1]
    h = a / jnp.maximum(d, 1.0) + z_ref[...]
    if relu:
        h = jnp.maximum(h, 0.0)
    o_ref[...] = h


def _combine(acc, deg, z, relu):
    return pl.pallas_call(
        functools.partial(_combine_body, relu),
        grid=(N // BLK,),
        in_specs=[
            pl.BlockSpec((NC, BLK, D), lambda i: (0, i, 0)),
            pl.BlockSpec((NC, BLK, D), lambda i: (0, i, 0)),
            pl.BlockSpec((BLK, D), lambda i: (i, 0)),
        ],
        out_specs=pl.BlockSpec((BLK, D), lambda i: (i, 0)),
        out_shape=jax.ShapeDtypeStruct((N, D), jnp.float32),
    )(acc, deg, z)


def kernel(x, edge_index, W1_l, b1, W1_r, W2_l, b2, W2_r):
    src = edge_index[0]
    dst = edge_index[1]
    dst3 = dst.reshape(NW, NCHUNK, CH)

    sc_feat = _make_sc_feat()
    sc_deg = _make_sc_deg()

    deg = sc_deg(dst3)
    y1, z1 = _mm2(x, W1_l.T, W1_r.T, b1)
    acc1 = sc_feat(src, dst, y1)
    y2, z2 = _combine_mm(acc1, deg, z1, W2_l.T, W2_r.T, b2)
    acc2 = sc_feat(src, dst, y2)
    out = _combine(acc2, deg, z2, False)
    return out


# speedup vs baseline: 10.4448x; 1.0006x over previous
import functools

import jax
import jax.numpy as jnp
from jax import lax
from jax.experimental import pallas as pl
from jax.experimental.pallas import tpu as pltpu
from jax.experimental.pallas import tpu_sc as plsc

N = 10000
E = 320000
D = 128
DEGW = 16
NC = 2
NS = 16
NW = NC * NS
EPT = E // NW
CH = 80
NCHUNK = EPT // CH
RPT = 624
TAIL = N - NS * RPT


def _zero_rows(ref, nrows, ncol16):
    def body(i, c):
        r = i // ncol16
        k = i % ncol16
        ref[r, pl.ds(k * 16, 16)] = jnp.zeros((16,), jnp.float32)
        return c
    lax.fori_loop(0, nrows * ncol16, body, 0)


def _fill_spmem_zero(zblk, sp, rbase, sid):
    nfull = RPT // CH
    rem = RPT - nfull * CH
    for k in range(nfull):
        pltpu.sync_copy(zblk, sp.at[pl.ds(rbase + k * CH, CH)])
    if rem:
        pltpu.sync_copy(zblk.at[pl.ds(0, rem)],
                        sp.at[pl.ds(rbase + nfull * CH, rem)])
    @pl.when(sid == NS - 1)
    def _():
        pltpu.sync_copy(zblk.at[pl.ds(0, TAIL)],
                        sp.at[pl.ds(NS * RPT, TAIL)])


def _sc_feat_body(src_hbm, dst_hbm, y_hbm, acc_out,
                  sidx_a, sidx_b, sidx_c, didx_a, didx_b, didx_c,
                  rows_a, rows_b, rows_c, acc_sp, semg, semsi, semdi, semsc):
    cid = lax.axis_index("c")
    sid = lax.axis_index("s")
    wid = sid * NC + cid
    rbase = sid * RPT
    ebase = wid * EPT

    _zero_rows(rows_a, CH, D // 16)
    _fill_spmem_zero(rows_a, acc_sp, rbase, sid)
    plsc.subcore_barrier()

    rows = (rows_a, rows_b, rows_c)
    sib = (sidx_a, sidx_b, sidx_c)
    dib = (didx_a, didx_b, didx_c)

    def si_load(jj, sl):
        return pltpu.async_copy(
            src_hbm.at[pl.ds(ebase + jj * CH, CH)], sl, semsi)

    def di_load(jj, dl):
        return pltpu.async_copy(
            dst_hbm.at[pl.ds(ebase + jj * CH, CH)], dl, semdi)

    pltpu.sync_copy(src_hbm.at[pl.ds(ebase, CH)], sidx_a)
    pltpu.async_copy(y_hbm.at[sidx_a], rows_a, semg)
    pltpu.sync_copy(src_hbm.at[pl.ds(ebase + CH, CH)], sidx_b)
    pltpu.async_copy(y_hbm.at[sidx_b], rows_b, semg)
    si_load(2, sidx_c)
    di_load(0, didx_a)

    @pl.loop(0, NCHUNK + 2 - ((NCHUNK + 2) % 3), step=3)
    def _(j):
        for b0 in range(3):
            jj = j + b0
            b = b0

            @pl.when(jj < NCHUNK)
            def _():
                pltpu.make_async_copy(y_hbm.at[sib[b]], rows[b], semg).wait()

                def wait_sc_prev():
                    pltpu.make_async_copy(
                        rows[(b + 2) % 3], acc_sp.at[dib[(b + 2) % 3]],
                        semsc).wait()
                if b0 == 0:
                    @pl.when(jj > 0)
                    def _():
                        wait_sc_prev()
                else:
                    wait_sc_prev()

                @pl.when(jj + 2 < NCHUNK)
                def _():
                    pltpu.make_async_copy(
                        src_hbm.at[pl.ds(ebase + (jj + 2) * CH, CH)],
                        sib[(b + 2) % 3], semsi).wait()
                    pltpu.async_copy(y_hbm.at[sib[(b + 2) % 3]],
                                     rows[(b + 2) % 3], semg)

                    @pl.when(jj + 3 < NCHUNK)
                    def _():
                        si_load(jj + 3, sib[b])

                @pl.when(jj + 1 < NCHUNK)
                def _():
                    di_load(jj + 1, dib[(b + 1) % 3])

                pltpu.make_async_copy(
                    dst_hbm.at[pl.ds(ebase + jj * CH, CH)],
                    dib[b], semdi).wait()
                pltpu.async_copy(rows[b], acc_sp.at[dib[b]], semsc, add=True)

    pltpu.make_async_copy(
        rows[(NCHUNK - 1) % 3], acc_sp.at[dib[(NCHUNK - 1) % 3]], semsc).wait()
    plsc.subcore_barrier()

    pltpu.sync_copy(acc_sp.at[pl.ds(rbase, RPT)],
                    acc_out.at[cid, pl.ds(rbase, RPT)])
    @pl.when(sid == NS - 1)
    def _():
        pltpu.sync_copy(acc_sp.at[pl.ds(NS * RPT, TAIL)],
                        acc_out.at[cid, pl.ds(NS * RPT, TAIL)])


def _make_sc_feat():
    return pl.kernel(
        _sc_feat_body,
        out_type=jax.ShapeDtypeStruct((NC, N, D), jnp.float32),
        scratch_types=[
            pltpu.VMEM((CH,), jnp.int32),
            pltpu.VMEM((CH,), jnp.int32),
            pltpu.VMEM((CH,), jnp.int32),
            pltpu.VMEM((CH,), jnp.int32),
            pltpu.VMEM((CH,), jnp.int32),
            pltpu.VMEM((CH,), jnp.int32),
            pltpu.VMEM((CH, D), jnp.float32),
            pltpu.VMEM((CH, D), jnp.float32),
            pltpu.VMEM((CH, D), jnp.float32),
            pltpu.VMEM_SHARED((N, D), jnp.float32),
            pltpu.SemaphoreType.DMA,
            pltpu.SemaphoreType.DMA,
            pltpu.SemaphoreType.DMA,
            pltpu.SemaphoreType.DMA,
        ],
        mesh=plsc.VectorSubcoreMesh(core_axis_name="c", subcore_axis_name="s"),
    )


DW = 128


def _sc_deg_body(dst_hbm, deg_out, didx_all, ones, deg_sp, semsc):
    cid = lax.axis_index("c")
    sid = lax.axis_index("s")
    wid = sid * NC + cid
    rbase = sid * RPT

    _zero_rows(ones, CH, DW // 16)
    _fill_spmem_zero(ones, deg_sp, rbase, sid)
    def fill1(i, c):
        r = i // (DW // 16)
        k = i % (DW // 16)
        ones[r, pl.ds(k * 16, 16)] = jnp.ones((16,), jnp.float32)
        return c
    lax.fori_loop(0, CH * (DW // 16), fill1, 0)
    pltpu.sync_copy(dst_hbm.at[wid], didx_all)
    plsc.subcore_barrier()

    @pl.loop(0, NCHUNK)
    def _(j):
        @pl.when(j >= 8)
        def _():
            pltpu.make_async_copy(ones, deg_sp.at[didx_all.at[j - 8]],
                                  semsc).wait()
        pltpu.async_copy(ones, deg_sp.at[didx_all.at[j]], semsc, add=True)

    for k in range(8):
        pltpu.make_async_copy(ones, deg_sp.at[didx_all.at[NCHUNK - 8 + k]],
                              semsc).wait()
    plsc.subcore_barrier()
    pltpu.sync_copy(deg_sp.at[pl.ds(rbase, RPT)],
                    deg_out.at[cid, pl.ds(rbase, RPT)])
    @pl.when(sid == NS - 1)
    def _():
        pltpu.sync_copy(deg_sp.at[pl.ds(NS * RPT, TAIL)],
                        deg_out.at[cid, pl.ds(NS * RPT, TAIL)])


def _make_sc_deg():
    return pl.kernel(
        _sc_deg_body,
        out_type=jax.ShapeDtypeStruct((NC, N, DW), jnp.float32),
        scratch_types=[
            pltpu.VMEM((NCHUNK, CH), jnp.int32),
            pltpu.VMEM((CH, DW), jnp.float32),
            pltpu.VMEM_SHARED((N, DW), jnp.float32),
            pltpu.SemaphoreType.DMA,
        ],
        mesh=plsc.VectorSubcoreMesh(core_axis_name="c", subcore_axis_name="s"),
    )


BLK = 400


def _mm2_body(x_ref, wlT_ref, wrT_ref, b_ref, y_ref, z_ref):
    xb = x_ref[...]
    y_ref[...] = jnp.dot(xb, wlT_ref[...], preferred_element_type=jnp.float32)
    z_ref[...] = (jnp.dot(xb, wrT_ref[...], preferred_element_type=jnp.float32)
                  + b_ref[...])


def _mm2(x, WlT, WrT, b):
    return pl.pallas_call(
        _mm2_body,
        grid=(N // BLK,),
        in_specs=[
            pl.BlockSpec((BLK, D), lambda i: (i, 0)),
            pl.BlockSpec((D, D), lambda i: (0, 0)),
            pl.BlockSpec((D, D), lambda i: (0, 0)),
            pl.BlockSpec((1, D), lambda i: (0, 0)),
        ],
        out_specs=[pl.BlockSpec((BLK, D), lambda i: (i, 0)),
                   pl.BlockSpec((BLK, D), lambda i: (i, 0))],
        out_shape=[jax.ShapeDtypeStruct((N, D), jnp.float32),
                   jax.ShapeDtypeStruct((N, D), jnp.float32)],
    )(x, WlT, WrT, b.reshape(1, D))


def _combine_mm_body(acc_ref, deg_ref, z_ref, wlT_ref, wrT_ref, b_ref,
                     y_ref, z2_ref):
    a = acc_ref[0] + acc_ref[1]
    dg = deg_ref[0] + deg_ref[1]
    h = a / jnp.maximum(dg[:, 0:1], 1.0) + z_ref[...]
    h = jnp.maximum(h, 0.0)
    y_ref[...] = jnp.dot(h, wlT_ref[...], preferred_element_type=jnp.float32)
    z2_ref[...] = (jnp.dot(h, wrT_ref[...], preferred_element_type=jnp.float32)
                   + b_ref[...])


def _combine_mm(acc, deg, z, WlT, WrT, b):
    return pl.pallas_call(
        _combine_mm_body,
        grid=(N // BLK,),
        in_specs=[
            pl.BlockSpec((NC, BLK, D), lambda i: (0, i, 0)),
            pl.BlockSpec((NC, BLK, DW), lambda i: (0, i, 0)),
            pl.BlockSpec((BLK, D), lambda i: (i, 0)),
            pl.BlockSpec((D, D), lambda i: (0, 0)),
            pl.BlockSpec((D, D), lambda i: (0, 0)),
            pl.BlockSpec((1, D), lambda i: (0, 0)),
        ],
        out_specs=[pl.BlockSpec((BLK, D), lambda i: (i, 0)),
                   pl.BlockSpec((BLK, D), lambda i: (i, 0))],
        out_shape=[jax.ShapeDtypeStruct((N, D), jnp.float32),
                   jax.ShapeDtypeStruct((N, D), jnp.float32)],
    )(acc, deg, z, WlT, WrT, b.reshape(1, D))


def _combine_body(relu, acc_ref, deg_ref, z_ref, o_ref):
    a = acc_ref[0] + acc_ref[1]
    dg = deg_ref[0] + deg_ref[1]
    d = dg[:, 0:1]
    h = a / jnp.maximum(d, 1.0) + z_ref[...]
    if relu:
        h = jnp.maximum(h, 0.0)
    o_ref[...] = h


def _combine(acc, deg, z, relu):
    return pl.pallas_call(
        functools.partial(_combine_body, relu),
        grid=(N // BLK,),
        in_specs=[
            pl.BlockSpec((NC, BLK, D), lambda i: (0, i, 0)),
            pl.BlockSpec((NC, BLK, DW), lambda i: (0, i, 0)),
            pl.BlockSpec((BLK, D), lambda i: (i, 0)),
        ],
        out_specs=pl.BlockSpec((BLK, D), lambda i: (i, 0)),
        out_shape=jax.ShapeDtypeStruct((N, D), jnp.float32),
    )(acc, deg, z)


def kernel(x, edge_index, W1_l, b1, W1_r, W2_l, b2, W2_r):
    src = edge_index[0]
    dst = edge_index[1]
    dst3 = dst.reshape(NW, NCHUNK, CH)

    sc_feat = _make_sc_feat()
    sc_deg = _make_sc_deg()

    deg = sc_deg(dst3)
    y1, z1 = _mm2(x, W1_l.T, W1_r.T, b1)
    acc1 = sc_feat(src, dst, y1)
    y2, z2 = _combine_mm(acc1, deg, z1, W2_l.T, W2_r.T, b2)
    acc2 = sc_feat(src, dst, y2)
    out = _combine(acc2, deg, z2, False)
    return out


# speedup vs baseline: 10.4499x; 1.0005x over previous
import functools

import jax
import jax.numpy as jnp
from jax import lax
from jax.experimental import pallas as pl
from jax.experimental.pallas import tpu as pltpu
from jax.experimental.pallas import tpu_sc as plsc

N = 10000
E = 320000
D = 128
NC = 2
NS = 16
NW = NC * NS
EPT = E // NW
CH = 80
NCHUNK = EPT // CH
RPT = 624
TAIL = N - NS * RPT


def _zero_rows(ref, nrows, ncol16):
    def body(i, c):
        r = i // ncol16
        k = i % ncol16
        ref[r, pl.ds(k * 16, 16)] = jnp.zeros((16,), jnp.float32)
        return c
    lax.fori_loop(0, nrows * ncol16, body, 0)


def _fill_spmem_zero(zblk, sp, rbase, sid):
    nfull = RPT // CH
    rem = RPT - nfull * CH
    for k in range(nfull):
        pltpu.sync_copy(zblk, sp.at[pl.ds(rbase + k * CH, CH)])
    if rem:
        pltpu.sync_copy(zblk.at[pl.ds(0, rem)],
                        sp.at[pl.ds(rbase + nfull * CH, rem)])
    @pl.when(sid == NS - 1)
    def _():
        pltpu.sync_copy(zblk.at[pl.ds(0, TAIL)],
                        sp.at[pl.ds(NS * RPT, TAIL)])


def _sc_feat_body(src_hbm, dst_hbm, y_hbm, acc_out,
                  sidx_a, sidx_b, sidx_c, didx_a, didx_b, didx_c,
                  rows_a, rows_b, rows_c, acc_sp, semg, semsi, semdi, semsc):
    cid = lax.axis_index("c")
    sid = lax.axis_index("s")
    wid = sid * NC + cid
    rbase = sid * RPT
    ebase = wid * EPT

    _zero_rows(rows_a, CH, D // 16)
    _fill_spmem_zero(rows_a, acc_sp, rbase, sid)
    plsc.subcore_barrier()

    rows = (rows_a, rows_b, rows_c)
    sib = (sidx_a, sidx_b, sidx_c)
    dib = (didx_a, didx_b, didx_c)

    def si_load(jj, sl):
        return pltpu.async_copy(
            src_hbm.at[pl.ds(ebase + jj * CH, CH)], sl, semsi)

    def di_load(jj, dl):
        return pltpu.async_copy(
            dst_hbm.at[pl.ds(ebase + jj * CH, CH)], dl, semdi)

    pltpu.sync_copy(src_hbm.at[pl.ds(ebase, CH)], sidx_a)
    pltpu.async_copy(y_hbm.at[sidx_a], rows_a, semg)
    pltpu.sync_copy(src_hbm.at[pl.ds(ebase + CH, CH)], sidx_b)
    pltpu.async_copy(y_hbm.at[sidx_b], rows_b, semg)
    si_load(2, sidx_c)
    di_load(0, didx_a)

    @pl.loop(0, NCHUNK + 2 - ((NCHUNK + 2) % 3), step=3)
    def _(j):
        for b0 in range(3):
            jj = j + b0
            b = b0

            @pl.when(jj < NCHUNK)
            def _():
                pltpu.make_async_copy(y_hbm.at[sib[b]], rows[b], semg).wait()

                def wait_sc_prev():
                    pltpu.make_async_copy(
                        rows[(b + 2) % 3], acc_sp.at[dib[(b + 2) % 3]],
                        semsc).wait()
                if b0 == 0:
                    @pl.when(jj > 0)
                    def _():
                        wait_sc_prev()
                else:
                    wait_sc_prev()

                @pl.when(jj + 2 < NCHUNK)
                def _():
                    pltpu.make_async_copy(
                        src_hbm.at[pl.ds(ebase + (jj + 2) * CH, CH)],
                        sib[(b + 2) % 3], semsi).wait()
                    pltpu.async_copy(y_hbm.at[sib[(b + 2) % 3]],
                                     rows[(b + 2) % 3], semg)

                    @pl.when(jj + 3 < NCHUNK)
                    def _():
                        si_load(jj + 3, sib[b])

                @pl.when(jj + 1 < NCHUNK)
                def _():
                    di_load(jj + 1, dib[(b + 1) % 3])

                pltpu.make_async_copy(
                    dst_hbm.at[pl.ds(ebase + jj * CH, CH)],
                    dib[b], semdi).wait()
                pltpu.async_copy(rows[b], acc_sp.at[dib[b]], semsc, add=True)

    pltpu.make_async_copy(
        rows[(NCHUNK - 1) % 3], acc_sp.at[dib[(NCHUNK - 1) % 3]], semsc).wait()
    plsc.subcore_barrier()

    pltpu.sync_copy(acc_sp.at[pl.ds(rbase, RPT)],
                    acc_out.at[cid, pl.ds(rbase, RPT)])
    @pl.when(sid == NS - 1)
    def _():
        pltpu.sync_copy(acc_sp.at[pl.ds(NS * RPT, TAIL)],
                        acc_out.at[cid, pl.ds(NS * RPT, TAIL)])


def _make_sc_feat():
    return pl.kernel(
        _sc_feat_body,
        out_type=jax.ShapeDtypeStruct((NC, N, D), jnp.float32),
        scratch_types=[
            pltpu.VMEM((CH,), jnp.int32),
            pltpu.VMEM((CH,), jnp.int32),
            pltpu.VMEM((CH,), jnp.int32),
            pltpu.VMEM((CH,), jnp.int32),
            pltpu.VMEM((CH,), jnp.int32),
            pltpu.VMEM((CH,), jnp.int32),
            pltpu.VMEM((CH, D), jnp.float32),
            pltpu.VMEM((CH, D), jnp.float32),
            pltpu.VMEM((CH, D), jnp.float32),
            pltpu.VMEM_SHARED((N, D), jnp.float32),
            pltpu.SemaphoreType.DMA,
            pltpu.SemaphoreType.DMA,
            pltpu.SemaphoreType.DMA,
            pltpu.SemaphoreType.DMA,
        ],
        mesh=plsc.VectorSubcoreMesh(core_axis_name="c", subcore_axis_name="s"),
    )


DW = 128


def _sc_deg_body(dst_hbm, deg_out, didx_all, ones, deg_sp, semsc):
    cid = lax.axis_index("c")
    sid = lax.axis_index("s")
    wid = sid * NC + cid
    rbase = sid * RPT

    _zero_rows(ones, CH, DW // 16)
    _fill_spmem_zero(ones, deg_sp, rbase, sid)
    def fill1(i, c):
        r = i // (DW // 16)
        k = i % (DW // 16)
        ones[r, pl.ds(k * 16, 16)] = jnp.ones((16,), jnp.float32)
        return c
    lax.fori_loop(0, CH * (DW // 16), fill1, 0)
    pltpu.sync_copy(dst_hbm.at[wid], didx_all)
    plsc.subcore_barrier()

    @pl.loop(0, NCHUNK)
    def _(j):
        @pl.when(j >= 8)
        def _():
            pltpu.make_async_copy(ones, deg_sp.at[didx_all.at[j - 8]],
                                  semsc).wait()
        pltpu.async_copy(ones, deg_sp.at[didx_all.at[j]], semsc, add=True)

    for k in range(8):
        pltpu.make_async_copy(ones, deg_sp.at[didx_all.at[NCHUNK - 8 + k]],
                              semsc).wait()
    plsc.subcore_barrier()
    pltpu.sync_copy(deg_sp.at[pl.ds(rbase, RPT)],
                    deg_out.at[cid, pl.ds(rbase, RPT)])
    @pl.when(sid == NS - 1)
    def _():
        pltpu.sync_copy(deg_sp.at[pl.ds(NS * RPT, TAIL)],
                        deg_out.at[cid, pl.ds(NS * RPT, TAIL)])


def _make_sc_deg():
    return pl.kernel(
        _sc_deg_body,
        out_type=jax.ShapeDtypeStruct((NC, N, DW), jnp.float32),
        scratch_types=[
            pltpu.VMEM((NCHUNK, CH), jnp.int32),
            pltpu.VMEM((CH, DW), jnp.float32),
            pltpu.VMEM_SHARED((N, DW), jnp.float32),
            pltpu.SemaphoreType.DMA,
        ],
        mesh=plsc.VectorSubcoreMesh(core_axis_name="c", subcore_axis_name="s"),
    )


BLK = 400


def _mm2_body(x_ref, wlT_ref, wrT_ref, b_ref, y_ref, z_ref):
    xb = x_ref[...]
    y_ref[...] = jnp.dot(xb, wlT_ref[...], preferred_element_type=jnp.float32)
    z_ref[...] = (jnp.dot(xb, wrT_ref[...], preferred_element_type=jnp.float32)
                  + b_ref[...])


def _mm2(x, WlT, WrT, b):
    return pl.pallas_call(
        _mm2_body,
        grid=(N // BLK,),
        in_specs=[
            pl.BlockSpec((BLK, D), lambda i: (i, 0)),
            pl.BlockSpec((D, D), lambda i: (0, 0)),
            pl.BlockSpec((D, D), lambda i: (0, 0)),
            pl.BlockSpec((1, D), lambda i: (0, 0)),
        ],
        out_specs=[pl.BlockSpec((BLK, D), lambda i: (i, 0)),
                   pl.BlockSpec((BLK, D), lambda i: (i, 0))],
        out_shape=[jax.ShapeDtypeStruct((N, D), jnp.float32),
                   jax.ShapeDtypeStruct((N, D), jnp.float32)],
    )(x, WlT, WrT, b.reshape(1, D))


def _combine_mm_body(acc_ref, deg_ref, z_ref, wlT_ref, wrT_ref, b_ref,
                     y_ref, z2_ref):
    a = acc_ref[0] + acc_ref[1]
    dg = deg_ref[0] + deg_ref[1]
    h = a / jnp.maximum(dg[:, 0:1], 1.0) + z_ref[...]
    h = jnp.maximum(h, 0.0)
    y_ref[...] = jnp.dot(h, wlT_ref[...], preferred_element_type=jnp.float32)
    z2_ref[...] = (jnp.dot(h, wrT_ref[...], preferred_element_type=jnp.float32)
                   + b_ref[...])


def _combine_mm(acc, deg, z, WlT, WrT, b):
    return pl.pallas_call(
        _combine_mm_body,
        grid=(N // BLK,),
        in_specs=[
            pl.BlockSpec((NC, BLK, D), lambda i: (0, i, 0)),
            pl.BlockSpec((NC, BLK, DW), lambda i: (0, i, 0)),
            pl.BlockSpec((BLK, D), lambda i: (i, 0)),
            pl.BlockSpec((D, D), lambda i: (0, 0)),
            pl.BlockSpec((D, D), lambda i: (0, 0)),
            pl.BlockSpec((1, D), lambda i: (0, 0)),
        ],
        out_specs=[pl.BlockSpec((BLK, D), lambda i: (i, 0)),
                   pl.BlockSpec((BLK, D), lambda i: (i, 0))],
        out_shape=[jax.ShapeDtypeStruct((N, D), jnp.float32),
                   jax.ShapeDtypeStruct((N, D), jnp.float32)],
    )(acc, deg, z, WlT, WrT, b.reshape(1, D))


def _combine_body(relu, acc_ref, deg_ref, z_ref, o_ref):
    a = acc_ref[0] + acc_ref[1]
    dg = deg_ref[0] + deg_ref[1]
    d = dg[:, 0:1]
    h = a / jnp.maximum(d, 1.0) + z_ref[...]
    if relu:
        h = jnp.maximum(h, 0.0)
    o_ref[...] = h


def _combine(acc, deg, z, relu):
    return pl.pallas_call(
        functools.partial(_combine_body, relu),
        grid=(N // BLK,),
        in_specs=[
            pl.BlockSpec((NC, BLK, D), lambda i: (0, i, 0)),
            pl.BlockSpec((NC, BLK, DW), lambda i: (0, i, 0)),
            pl.BlockSpec((BLK, D), lambda i: (i, 0)),
        ],
        out_specs=pl.BlockSpec((BLK, D), lambda i: (i, 0)),
        out_shape=jax.ShapeDtypeStruct((N, D), jnp.float32),
    )(acc, deg, z)


def kernel(x, edge_index, W1_l, b1, W1_r, W2_l, b2, W2_r):
    src = edge_index[0]
    dst = edge_index[1]
    dst3 = dst.reshape(NW, NCHUNK, CH)

    sc_feat = _make_sc_feat()
    sc_deg = _make_sc_deg()

    deg = sc_deg(dst3)
    y1, z1 = _mm2(x, W1_l.T, W1_r.T, b1)
    acc1 = sc_feat(src, dst, y1)
    y2, z2 = _combine_mm(acc1, deg, z1, W2_l.T, W2_r.T, b2)
    acc2 = sc_feat(src, dst, y2)
    out = _combine(acc2, deg, z2, False)
    return out
